# Initial kernel scaffold; baseline (speedup 1.0000x reference)
import jax
import jax.numpy as jnp
from jax.experimental import pallas as pl


def kernel(x, edge_index, batch, Wp, bp, W1, as1, ad1, b1, W2, as2, ad2, b2, W3, as3, ad3, b3, W4, as4, ad4, b4, g1, be1, g2, be2, g3, be3, g4, be4, Wf, bf):
    raise NotImplementedError("write your pallas kernel here")


# speedup vs baseline: 18.1848x; 18.1848x over previous
import functools

import jax
import jax.numpy as jnp
from jax import lax
from jax.experimental import pallas as pl
from jax.experimental.pallas import tpu as pltpu
from jax.experimental.pallas import tpu_sc as plsc

N_NODES = 50000
E_EDGES = 800000
G_SEG = 10
NT = 64
D_T = 784
NPAD = NT * D_T
K = 128
BR = 784
F32 = jnp.float32
I32 = jnp.int32


def _iota16():
    return lax.iota(I32, 16)


def _take16(v, idx):
    dnums = lax.GatherDimensionNumbers(
        offset_dims=(), collapsed_slice_dims=(0,), start_index_map=(0,))
    return lax.gather(v, idx[:, None], dimension_numbers=dnums,
                      slice_sizes=(1,),
                      mode=lax.GatherScatterMode.PROMISE_IN_BOUNDS)


def _seg_masks(dstv):
    io = _iota16()
    sames = []
    for k in (1, 2, 4, 8):
        prev = _take16(dstv, jnp.maximum(io - k, 0))
        sames.append((io >= k) & (prev == dstv))
    nxt = _take16(dstv, jnp.minimum(io + 1, 15))
    end = (dstv != nxt) | (io == 15)
    return sames, end


def _seg_cummax(v, sames):
    io = _iota16()
    for k, same in zip((1, 2, 4, 8), sames):
        sh = _take16(v, jnp.maximum(io - k, 0))
        v = jnp.where(same, jnp.maximum(v, sh), v)
    return v


def _seg_cumsum(v, sames):
    io = _iota16()
    for k, same in zip((1, 2, 4, 8), sames):
        sh = _take16(v, jnp.maximum(io - k, 0))
        v = v + jnp.where(same, sh, jnp.zeros_like(v))
    return v


def _splat(val):
    return _iota16() * 0 + val


def _make_gat_sc(H, OC):
    mesh = plsc.VectorSubcoreMesh(core_axis_name="c", subcore_axis_name="s")

    @functools.partial(
        pl.kernel,
        mesh=mesh,
        compiler_params=pltpu.CompilerParams(
            needs_layout_passes=False, use_tc_tiling_on_sc=False),
        out_type=jax.ShapeDtypeStruct((NPAD, 64), F32),
        scratch_types=[
            pltpu.VMEM((K,), I32),
            pltpu.VMEM((K,), I32),
            pltpu.VMEM((K, H), F32),
            pltpu.VMEM((K, 64), F32),
            pltpu.VMEM((D_T, H), F32),
            pltpu.VMEM((D_T, H), F32),
            pltpu.VMEM((D_T, H), F32),
            pltpu.VMEM((D_T, 64), F32),
            pltpu.VMEM((16,), I32),
            pltpu.SemaphoreType.DMA,
        ],
    )
    def gat_sc(h_hbm, as_hbm, ad_hbm, srcs_hbm, dsts_hbm, offs_hbm,
               zero64_hbm, zeroh_hbm, neg_hbm, out_hbm,
               srcc, dstc, asc, hc, adl, amax, den, acc, offr, sem):
        wid = lax.axis_index("s") * 2 + lax.axis_index("c")
        io = _iota16()
        for vt in range(2):
            r = wid * 2 + vt
            d0 = r * D_T
            pltpu.sync_copy(offs_hbm.at[r], offr)
            row = offr[...]
            e0 = row[0]
            e1 = row[1]
            e0a = (e0 // 8) * 8
            nch = (e1 - e0a + (K - 1)) // K
            pltpu.sync_copy(ad_hbm.at[pl.ds(d0, D_T)], adl)
            pltpu.sync_copy(neg_hbm, amax)
            pltpu.sync_copy(zeroh_hbm, den)
            pltpu.sync_copy(zero64_hbm, acc)

            def stage(base, with_h):
                pltpu.sync_copy(srcs_hbm.at[pl.ds(base, K)], srcc)
                pltpu.sync_copy(dsts_hbm.at[pl.ds(base, K)], dstc)
                pltpu.async_copy(as_hbm.at[srcc], asc, sem).wait()
                if with_h:
                    pltpu.async_copy(h_hbm.at[srcc], hc, sem).wait()

            def group_common(base, g):
                lane_e = base + g * 16 + io
                valid = (lane_e >= e0) & (lane_e < e1)
                dstv = dstc[pl.ds(g * 16, 16)] - d0
                rowc = jnp.clip(dstv, 0, D_T - 1)
                rowa = g * 16 + io
                sames, end = _seg_masks(dstv)
                return valid, rowc, rowa, sames, end & valid

            def alpha_head(rowa, rowc, h):
                colh = _splat(h)
                a_sv = plsc.load_gather(asc, [rowa, colh])
                a_dv = plsc.load_gather(adl, [rowc, colh])
                al = a_sv + a_dv
                return jnp.where(al > 0, al, 0.2 * al), colh

            def p1_group(g, base):
                valid, rowc, rowa, sames, smask = group_common(base, g)
                for h in range(H):
                    al, colh = alpha_head(rowa, rowc, h)
                    al = jnp.where(valid, al, jnp.full((16,), -1e30, F32))
                    sm = _seg_cummax(al, sames)
                    cur = plsc.load_gather(amax, [rowc, colh])
                    plsc.store_scatter(amax, [rowc, colh],
                                       jnp.maximum(cur, sm), mask=smask)
                return base

            def p1_chunk(c, _):
                base = e0a + c * K
                stage(base, False)
                lax.fori_loop(0, 8, p1_group, base)
                return 0

            lax.fori_loop(0, nch, p1_chunk, 0)

            def p2_group(g, base):
                valid, rowc, rowa, sames, smask = group_common(base, g)
                for h in range(H):
                    al, colh = alpha_head(rowa, rowc, h)
                    mx = plsc.load_gather(amax, [rowc, colh])
                    ex = jnp.where(valid, jnp.exp(al - mx),
                                   jnp.zeros((16,), F32))
                    ss = _seg_cumsum(ex, sames)
                    cur = plsc.load_gather(den, [rowc, colh])
                    plsc.store_scatter(den, [rowc, colh], cur + ss, mask=smask)
                return base

            def p2_chunk(c, _):
                base = e0a + c * K
                stage(base, False)
                lax.fori_loop(0, 8, p2_group, base)
                return 0

            lax.fori_loop(0, nch, p2_chunk, 0)

            def p3_group(g, base):
                valid, rowc, rowa, sames, smask = group_common(base, g)
                ws = []
                for h in range(H):
                    al, colh = alpha_head(rowa, rowc, h)
                    mx = plsc.load_gather(amax, [rowc, colh])
                    dn = plsc.load_gather(den, [rowc, colh])
                    w = jnp.exp(al - mx) / (dn + 1e-16)
                    ws.append(jnp.where(valid, w, jnp.zeros((16,), F32)))
                for f in range(64):
                    colf = _splat(f)
                    hv = plsc.load_gather(hc, [rowa, colf])
                    ss = _seg_cumsum(hv * ws[f // OC], sames)
                    cur = plsc.load_gather(acc, [rowc, colf])
                    plsc.store_scatter(acc, [rowc, colf], cur + ss, mask=smask)
                return base

            def p3_chunk(c, _):
                base = e0a + c * K
                stage(base, True)
                lax.fori_loop(0, 8, p3_group, base)
                return 0

            lax.fori_loop(0, nch, p3_chunk, 0)

            pltpu.sync_copy(acc, out_hbm.at[pl.ds(d0, D_T)])

    return gat_sc


def _ln_relu(y, g, be):
    m = jnp.mean(y, axis=-1, keepdims=True)
    v = jnp.mean((y - m) ** 2, axis=-1, keepdims=True)
    return jnp.maximum((y - m) / jnp.sqrt(v + 1e-5) * g + be, 0.0)


def _tc_pre_body(x_ref, wp_ref, bp_ref, w_ref, am_s_ref, am_d_ref,
                 hw_ref, as_ref, ad_ref):
    h0 = jnp.dot(x_ref[...], wp_ref[...],
                 preferred_element_type=F32) + bp_ref[...]
    hw = jnp.dot(h0, w_ref[...], preferred_element_type=F32)
    hw_ref[...] = hw
    as_ref[...] = jnp.dot(hw, am_s_ref[...], preferred_element_type=F32)
    ad_ref[...] = jnp.dot(hw, am_d_ref[...], preferred_element_type=F32)


def _tc_mid_body(o_ref, b_ref, g_ref, be_ref, w_ref, am_s_ref, am_d_ref,
                 hw_ref, as_ref, ad_ref):
    t = _ln_relu(o_ref[...] + b_ref[...], g_ref[...], be_ref[...])
    hw = jnp.dot(t, w_ref[...], preferred_element_type=F32)
    hw_ref[...] = hw
    as_ref[...] = jnp.dot(hw, am_s_ref[...], preferred_element_type=F32)
    ad_ref[...] = jnp.dot(hw, am_d_ref[...], preferred_element_type=F32)


def _full_spec(shape):
    return pl.BlockSpec(shape, lambda i: (0,) * len(shape))


def _row_spec(h, w):
    return pl.BlockSpec((h, w), lambda i: (i, 0))


def _tc_pre(x, wp, bp, w, ams, amd, hcols):
    grid = (NPAD // BR,)
    return pl.pallas_call(
        _tc_pre_body,
        grid=grid,
        in_specs=[_row_spec(BR, x.shape[1]), _full_spec(wp.shape),
                  _full_spec(bp.shape), _full_spec(w.shape),
                  _full_spec(ams.shape), _full_spec(amd.shape)],
        out_specs=[_row_spec(BR, 64), _row_spec(BR, hcols),
                   _row_spec(BR, hcols)],
        out_shape=[jax.ShapeDtypeStruct((NPAD, 64), F32),
                   jax.ShapeDtypeStruct((NPAD, hcols), F32),
                   jax.ShapeDtypeStruct((NPAD, hcols), F32)],
    )(x, wp, bp, w, ams, amd)


def _tc_mid(o, b, g, be, w, ams, amd, hcols):
    grid = (NPAD // BR,)
    return pl.pallas_call(
        _tc_mid_body,
        grid=grid,
        in_specs=[_row_spec(BR, 64), _full_spec(b.shape), _full_spec(g.shape),
                  _full_spec(be.shape), _full_spec(w.shape),
                  _full_spec(ams.shape), _full_spec(amd.shape)],
        out_specs=[_row_spec(BR, 64), _row_spec(BR, hcols),
                   _row_spec(BR, hcols)],
        out_shape=[jax.ShapeDtypeStruct((NPAD, 64), F32),
                   jax.ShapeDtypeStruct((NPAD, hcols), F32),
                   jax.ShapeDtypeStruct((NPAD, hcols), F32)],
    )(o, b, g, be, w, ams, amd)


def _tc_pool_body(o_ref, b_ref, g_ref, be_ref, bat_ref, pool_ref):
    i = pl.program_id(0)

    @pl.when(i == 0)
    def _():
        pool_ref[...] = jnp.full((16, 64), -1e30, F32)

    t = _ln_relu(o_ref[...] + b_ref[...], g_ref[...], be_ref[...])
    bat = bat_ref[...]
    for g in range(G_SEG):
        m = jnp.where(bat == g, t, jnp.full_like(t, -1e30))
        mg = jnp.max(m, axis=0, keepdims=True)
        pool_ref[pl.ds(g, 1), :] = jnp.maximum(pool_ref[pl.ds(g, 1), :], mg)

    @pl.when(i == NPAD // BR - 1)
    def _():
        rid = lax.broadcasted_iota(I32, (16, 64), 0)
        pool_ref[...] = jnp.where(rid < G_SEG, pool_ref[...],
                                  jnp.zeros((16, 64), F32))


def _tc_pool(o, b, g, be, bat):
    return pl.pallas_call(
        _tc_pool_body,
        grid=(NPAD // BR,),
        in_specs=[_row_spec(BR, 64), _full_spec(b.shape), _full_spec(g.shape),
                  _full_spec(be.shape), _row_spec(BR, 1)],
        out_specs=_full_spec((16, 64)),
        out_shape=jax.ShapeDtypeStruct((16, 64), F32),
    )(o, b, g, be, bat)


def _tc_final_body(p_ref, wf_ref, bf_ref, out_ref):
    v = jnp.dot(p_ref[...], wf_ref[...],
                preferred_element_type=F32) + bf_ref[...]
    n = jnp.sqrt(jnp.sum(v * v, axis=-1, keepdims=True))
    out_ref[...] = v / jnp.maximum(n, 1e-12)


def _tc_final(p, wf, bf):
    return pl.pallas_call(
        _tc_final_body,
        grid=(1,),
        in_specs=[_full_spec((16, 64)), _full_spec(wf.shape),
                  _full_spec(bf.shape)],
        out_specs=_full_spec((16, 128)),
        out_shape=jax.ShapeDtypeStruct((16, 128), F32),
    )(p, wf, bf)


def _amat(a, H, OC):
    m = jnp.zeros((H * OC, H), F32)
    return m.at[jnp.arange(H * OC), jnp.arange(H * OC) // OC].set(
        a.reshape(-1))


_GAT8 = _make_gat_sc(8, 8)
_GAT1 = _make_gat_sc(1, 64)


def kernel(x, edge_index, batch, Wp, bp, W1, as1, ad1, b1, W2, as2, ad2, b2,
           W3, as3, ad3, b3, W4, as4, ad4, b4, g1, be1, g2, be2, g3, be3,
           g4, be4, Wf, bf):
    loop = jnp.arange(N_NODES, dtype=edge_index.dtype)
    src = jnp.concatenate([edge_index[0], loop])
    dst = jnp.concatenate([edge_index[1], loop])
    order = jnp.argsort(dst)
    srcs = jnp.take(src, order)
    dsts = jnp.take(dst, order)
    offs = jnp.searchsorted(
        dsts, jnp.arange(NT + 1, dtype=I32) * D_T).astype(I32)
    srcs_p = jnp.concatenate([srcs, jnp.zeros((K,), I32)])
    dsts_p = jnp.concatenate([dsts, jnp.zeros((K,), I32)])
    offs2 = jnp.zeros((NT, 16), I32)
    offs2 = offs2.at[:, 0].set(offs[:-1]).at[:, 1].set(offs[1:])

    zero64 = jnp.zeros((D_T, 64), F32)
    zero8 = jnp.zeros((D_T, 8), F32)
    neg8 = jnp.full((D_T, 8), -1e30, F32)
    zero1 = jnp.zeros((D_T, 1), F32)
    neg1 = jnp.full((D_T, 1), -1e30, F32)

    x_p = jnp.pad(x, ((0, NPAD - N_NODES), (0, 0)))
    bat_p = jnp.pad(batch, (0, NPAD - N_NODES),
                    constant_values=G_SEG).reshape(NPAD, 1)

    hw, a_s, a_d = _tc_pre(x_p, Wp, bp.reshape(1, 64), W1,
                           _amat(as1, 8, 8), _amat(ad1, 8, 8), 8)
    o1 = _GAT8(hw, a_s, a_d, srcs_p, dsts_p, offs2, zero64, zero8, neg8)
    hw, a_s, a_d = _tc_mid(o1, b1.reshape(1, 64), g1.reshape(1, 64),
                           be1.reshape(1, 64), W2,
                           _amat(as2, 8, 8), _amat(ad2, 8, 8), 8)
    o2 = _GAT8(hw, a_s, a_d, srcs_p, dsts_p, offs2, zero64, zero8, neg8)
    hw, a_s, a_d = _tc_mid(o2, b2.reshape(1, 64), g2.reshape(1, 64),
                           be2.reshape(1, 64), W3,
                           _amat(as3, 8, 8), _amat(ad3, 8, 8), 8)
    o3 = _GAT8(hw, a_s, a_d, srcs_p, dsts_p, offs2, zero64, zero8, neg8)
    hw, a_s, a_d = _tc_mid(o3, b3.reshape(1, 64), g3.reshape(1, 64),
                           be3.reshape(1, 64), W4,
                           _amat(as4, 1, 64), _amat(ad4, 1, 64), 1)
    o4 = _GAT1(hw, a_s, a_d, srcs_p, dsts_p, offs2, zero64, zero1, neg1)
    pooled = _tc_pool(o4, b4.reshape(1, 64), g4.reshape(1, 64),
                      be4.reshape(1, 64), bat_p)
    v = _tc_final(pooled, Wf, bf.reshape(1, 128))
    return v[:G_SEG]


# speedup vs baseline: 19.2111x; 1.0564x over previous
import functools

import jax
import jax.numpy as jnp
from jax import lax
from jax.experimental import pallas as pl
from jax.experimental.pallas import tpu as pltpu
from jax.experimental.pallas import tpu_sc as plsc

N_NODES = 50000
E_EDGES = 800000
G_SEG = 10
NT = 64
D_T = 784
NPAD = NT * D_T
K = 128
BR = 784
F32 = jnp.float32
I32 = jnp.int32


def _iota16():
    return lax.iota(I32, 16)


def _take16(v, idx):
    dnums = lax.GatherDimensionNumbers(
        offset_dims=(), collapsed_slice_dims=(0,), start_index_map=(0,))
    return lax.gather(v, idx[:, None], dimension_numbers=dnums,
                      slice_sizes=(1,),
                      mode=lax.GatherScatterMode.PROMISE_IN_BOUNDS)


def _seg_masks(dstv):
    io = _iota16()
    sames = []
    for k in (1, 2, 4, 8):
        prev = _take16(dstv, jnp.maximum(io - k, 0))
        sames.append((io >= k) & (prev == dstv))
    nxt = _take16(dstv, jnp.minimum(io + 1, 15))
    end = (dstv != nxt) | (io == 15)
    return sames, end


def _seg_cummax(v, sames):
    io = _iota16()
    for k, same in zip((1, 2, 4, 8), sames):
        sh = _take16(v, jnp.maximum(io - k, 0))
        v = jnp.where(same, jnp.maximum(v, sh), v)
    return v


def _seg_cumsum(v, sames):
    io = _iota16()
    for k, same in zip((1, 2, 4, 8), sames):
        sh = _take16(v, jnp.maximum(io - k, 0))
        v = v + jnp.where(same, sh, jnp.zeros_like(v))
    return v


def _splat(val):
    return _iota16() * 0 + val


def _make_gat_sc(H, OC):
    mesh = plsc.VectorSubcoreMesh(core_axis_name="c", subcore_axis_name="s")

    @functools.partial(
        pl.kernel,
        mesh=mesh,
        compiler_params=pltpu.CompilerParams(
            needs_layout_passes=False, use_tc_tiling_on_sc=False),
        out_type=jax.ShapeDtypeStruct((NPAD, 64), F32),
        scratch_types=[
            pltpu.VMEM((K,), I32),
            pltpu.VMEM((K,), I32),
            pltpu.VMEM((K, H), F32),
            pltpu.VMEM((K, 64), F32),
            pltpu.VMEM((D_T, H), F32),
            pltpu.VMEM((D_T, H), F32),
            pltpu.VMEM((D_T, H), F32),
            pltpu.VMEM((D_T, 64), F32),
            pltpu.VMEM((16,), I32),
            pltpu.SemaphoreType.DMA,
        ],
    )
    def gat_sc(h_hbm, as_hbm, ad_hbm, srcs_hbm, dsts_hbm, offs_hbm,
               zero64_hbm, zeroh_hbm, neg_hbm, out_hbm,
               srcc, dstc, asc, hc, adl, amax, den, acc, offr, sem):
        wid = lax.axis_index("s") * 2 + lax.axis_index("c")
        io = _iota16()
        for vt in range(2):
            r = wid * 2 + vt
            d0 = r * D_T
            pltpu.sync_copy(offs_hbm.at[r], offr)
            row = offr[...]
            e0 = row[0]
            e1 = row[1]
            e0a = (e0 // 8) * 8
            nch = (e1 - e0a + (K - 1)) // K
            pltpu.sync_copy(ad_hbm.at[pl.ds(d0, D_T)], adl)
            pltpu.sync_copy(neg_hbm, amax)
            pltpu.sync_copy(zeroh_hbm, den)
            pltpu.sync_copy(zero64_hbm, acc)

            def stage(base, with_h):
                pltpu.sync_copy(srcs_hbm.at[pl.ds(base, K)], srcc)
                pltpu.sync_copy(dsts_hbm.at[pl.ds(base, K)], dstc)
                pltpu.async_copy(as_hbm.at[srcc], asc, sem).wait()
                if with_h:
                    pltpu.async_copy(h_hbm.at[srcc], hc, sem).wait()

            def group_common(base, g):
                lane_e = base + g * 16 + io
                valid = (lane_e >= e0) & (lane_e < e1)
                dstv = dstc[pl.ds(g * 16, 16)] - d0
                rowc = jnp.clip(dstv, 0, D_T - 1)
                rowa = g * 16 + io
                sames, end = _seg_masks(dstv)
                return valid, rowc, rowa, sames, end & valid

            def alpha_head(rowa, rowc, h):
                colh = _splat(h)
                a_sv = plsc.load_gather(asc, [rowa, colh])
                a_dv = plsc.load_gather(adl, [rowc, colh])
                al = a_sv + a_dv
                return jnp.where(al > 0, al, 0.2 * al), colh

            def p1_group(g, base):
                valid, rowc, rowa, sames, smask = group_common(base, g)
                for h in range(H):
                    al, colh = alpha_head(rowa, rowc, h)
                    al = jnp.where(valid, al, jnp.full((16,), -1e30, F32))
                    sm = _seg_cummax(al, sames)
                    cur = plsc.load_gather(amax, [rowc, colh])
                    plsc.store_scatter(amax, [rowc, colh],
                                       jnp.maximum(cur, sm), mask=smask)
                return base

            def p1_chunk(c, _):
                base = e0a + c * K
                stage(base, False)
                lax.fori_loop(0, 8, p1_group, base)
                return 0

            lax.fori_loop(0, nch, p1_chunk, 0)

            def p2_group(g, base):
                valid, rowc, rowa, sames, smask = group_common(base, g)
                exs = []
                for h in range(H):
                    al, colh = alpha_head(rowa, rowc, h)
                    mx = plsc.load_gather(amax, [rowc, colh])
                    ex = jnp.where(valid, jnp.exp(al - mx),
                                   jnp.zeros((16,), F32))
                    exs.append(ex)
                    ss = _seg_cumsum(ex, sames)
                    cur = plsc.load_gather(den, [rowc, colh])
                    plsc.store_scatter(den, [rowc, colh], cur + ss, mask=smask)
                for f in range(64):
                    colf = _splat(f)
                    hv = plsc.load_gather(hc, [rowa, colf])
                    ss = _seg_cumsum(hv * exs[f // OC], sames)
                    cur = plsc.load_gather(acc, [rowc, colf])
                    plsc.store_scatter(acc, [rowc, colf], cur + ss, mask=smask)
                return base

            def p2_chunk(c, _):
                base = e0a + c * K
                stage(base, True)
                lax.fori_loop(0, 8, p2_group, base)
                return 0

            lax.fori_loop(0, nch, p2_chunk, 0)

            def norm_row(rr, _):
                rsplat = _splat(rr)
                for k in range(4):
                    colf = k * 16 + io
                    dv = plsc.load_gather(den, [rsplat, colf // OC])
                    av = acc[rr, pl.ds(k * 16, 16)]
                    acc[rr, pl.ds(k * 16, 16)] = av / (dv + 1e-16)
                return 0

            lax.fori_loop(0, D_T, norm_row, 0)

            pltpu.sync_copy(acc, out_hbm.at[pl.ds(d0, D_T)])

    return gat_sc


def _ln_relu(y, g, be):
    m = jnp.mean(y, axis=-1, keepdims=True)
    v = jnp.mean((y - m) ** 2, axis=-1, keepdims=True)
    return jnp.maximum((y - m) / jnp.sqrt(v + 1e-5) * g + be, 0.0)


def _tc_pre_body(x_ref, wp_ref, bp_ref, w_ref, am_s_ref, am_d_ref,
                 hw_ref, as_ref, ad_ref):
    h0 = jnp.dot(x_ref[...], wp_ref[...],
                 preferred_element_type=F32) + bp_ref[...]
    hw = jnp.dot(h0, w_ref[...], preferred_element_type=F32)
    hw_ref[...] = hw
    as_ref[...] = jnp.dot(hw, am_s_ref[...], preferred_element_type=F32)
    ad_ref[...] = jnp.dot(hw, am_d_ref[...], preferred_element_type=F32)


def _tc_mid_body(o_ref, b_ref, g_ref, be_ref, w_ref, am_s_ref, am_d_ref,
                 hw_ref, as_ref, ad_ref):
    t = _ln_relu(o_ref[...] + b_ref[...], g_ref[...], be_ref[...])
    hw = jnp.dot(t, w_ref[...], preferred_element_type=F32)
    hw_ref[...] = hw
    as_ref[...] = jnp.dot(hw, am_s_ref[...], preferred_element_type=F32)
    ad_ref[...] = jnp.dot(hw, am_d_ref[...], preferred_element_type=F32)


def _full_spec(shape):
    return pl.BlockSpec(shape, lambda i: (0,) * len(shape))


def _row_spec(h, w):
    return pl.BlockSpec((h, w), lambda i: (i, 0))


def _tc_pre(x, wp, bp, w, ams, amd, hcols):
    grid = (NPAD // BR,)
    return pl.pallas_call(
        _tc_pre_body,
        grid=grid,
        in_specs=[_row_spec(BR, x.shape[1]), _full_spec(wp.shape),
                  _full_spec(bp.shape), _full_spec(w.shape),
                  _full_spec(ams.shape), _full_spec(amd.shape)],
        out_specs=[_row_spec(BR, 64), _row_spec(BR, hcols),
                   _row_spec(BR, hcols)],
        out_shape=[jax.ShapeDtypeStruct((NPAD, 64), F32),
                   jax.ShapeDtypeStruct((NPAD, hcols), F32),
                   jax.ShapeDtypeStruct((NPAD, hcols), F32)],
    )(x, wp, bp, w, ams, amd)


def _tc_mid(o, b, g, be, w, ams, amd, hcols):
    grid = (NPAD // BR,)
    return pl.pallas_call(
        _tc_mid_body,
        grid=grid,
        in_specs=[_row_spec(BR, 64), _full_spec(b.shape), _full_spec(g.shape),
                  _full_spec(be.shape), _full_spec(w.shape),
                  _full_spec(ams.shape), _full_spec(amd.shape)],
        out_specs=[_row_spec(BR, 64), _row_spec(BR, hcols),
                   _row_spec(BR, hcols)],
        out_shape=[jax.ShapeDtypeStruct((NPAD, 64), F32),
                   jax.ShapeDtypeStruct((NPAD, hcols), F32),
                   jax.ShapeDtypeStruct((NPAD, hcols), F32)],
    )(o, b, g, be, w, ams, amd)


def _tc_pool_body(o_ref, b_ref, g_ref, be_ref, bat_ref, pool_ref):
    i = pl.program_id(0)

    @pl.when(i == 0)
    def _():
        pool_ref[...] = jnp.full((16, 64), -1e30, F32)

    t = _ln_relu(o_ref[...] + b_ref[...], g_ref[...], be_ref[...])
    bat = bat_ref[...]
    for g in range(G_SEG):
        m = jnp.where(bat == g, t, jnp.full_like(t, -1e30))
        mg = jnp.max(m, axis=0, keepdims=True)
        pool_ref[pl.ds(g, 1), :] = jnp.maximum(pool_ref[pl.ds(g, 1), :], mg)

    @pl.when(i == NPAD // BR - 1)
    def _():
        rid = lax.broadcasted_iota(I32, (16, 64), 0)
        pool_ref[...] = jnp.where(rid < G_SEG, pool_ref[...],
                                  jnp.zeros((16, 64), F32))


def _tc_pool(o, b, g, be, bat):
    return pl.pallas_call(
        _tc_pool_body,
        grid=(NPAD // BR,),
        in_specs=[_row_spec(BR, 64), _full_spec(b.shape), _full_spec(g.shape),
                  _full_spec(be.shape), _row_spec(BR, 1)],
        out_specs=_full_spec((16, 64)),
        out_shape=jax.ShapeDtypeStruct((16, 64), F32),
    )(o, b, g, be, bat)


def _tc_final_body(p_ref, wf_ref, bf_ref, out_ref):
    v = jnp.dot(p_ref[...], wf_ref[...],
                preferred_element_type=F32) + bf_ref[...]
    n = jnp.sqrt(jnp.sum(v * v, axis=-1, keepdims=True))
    out_ref[...] = v / jnp.maximum(n, 1e-12)


def _tc_final(p, wf, bf):
    return pl.pallas_call(
        _tc_final_body,
        grid=(1,),
        in_specs=[_full_spec((16, 64)), _full_spec(wf.shape),
                  _full_spec(bf.shape)],
        out_specs=_full_spec((16, 128)),
        out_shape=jax.ShapeDtypeStruct((16, 128), F32),
    )(p, wf, bf)


def _amat(a, H, OC):
    m = jnp.zeros((H * OC, H), F32)
    return m.at[jnp.arange(H * OC), jnp.arange(H * OC) // OC].set(
        a.reshape(-1))


_GAT8 = _make_gat_sc(8, 8)
_GAT1 = _make_gat_sc(1, 64)


def kernel(x, edge_index, batch, Wp, bp, W1, as1, ad1, b1, W2, as2, ad2, b2,
           W3, as3, ad3, b3, W4, as4, ad4, b4, g1, be1, g2, be2, g3, be3,
           g4, be4, Wf, bf):
    loop = jnp.arange(N_NODES, dtype=edge_index.dtype)
    src = jnp.concatenate([edge_index[0], loop])
    dst = jnp.concatenate([edge_index[1], loop])
    order = jnp.argsort(dst)
    srcs = jnp.take(src, order)
    dsts = jnp.take(dst, order)
    offs = jnp.searchsorted(
        dsts, jnp.arange(NT + 1, dtype=I32) * D_T).astype(I32)
    srcs_p = jnp.concatenate([srcs, jnp.zeros((K,), I32)])
    dsts_p = jnp.concatenate([dsts, jnp.zeros((K,), I32)])
    offs2 = jnp.zeros((NT, 16), I32)
    offs2 = offs2.at[:, 0].set(offs[:-1]).at[:, 1].set(offs[1:])

    zero64 = jnp.zeros((D_T, 64), F32)
    zero8 = jnp.zeros((D_T, 8), F32)
    neg8 = jnp.full((D_T, 8), -1e30, F32)
    zero1 = jnp.zeros((D_T, 1), F32)
    neg1 = jnp.full((D_T, 1), -1e30, F32)

    x_p = jnp.pad(x, ((0, NPAD - N_NODES), (0, 0)))
    bat_p = jnp.pad(batch, (0, NPAD - N_NODES),
                    constant_values=G_SEG).reshape(NPAD, 1)

    hw, a_s, a_d = _tc_pre(x_p, Wp, bp.reshape(1, 64), W1,
                           _amat(as1, 8, 8), _amat(ad1, 8, 8), 8)
    o1 = _GAT8(hw, a_s, a_d, srcs_p, dsts_p, offs2, zero64, zero8, neg8)
    hw, a_s, a_d = _tc_mid(o1, b1.reshape(1, 64), g1.reshape(1, 64),
                           be1.reshape(1, 64), W2,
                           _amat(as2, 8, 8), _amat(ad2, 8, 8), 8)
    o2 = _GAT8(hw, a_s, a_d, srcs_p, dsts_p, offs2, zero64, zero8, neg8)
    hw, a_s, a_d = _tc_mid(o2, b2.reshape(1, 64), g2.reshape(1, 64),
                           be2.reshape(1, 64), W3,
                           _amat(as3, 8, 8), _amat(ad3, 8, 8), 8)
    o3 = _GAT8(hw, a_s, a_d, srcs_p, dsts_p, offs2, zero64, zero8, neg8)
    hw, a_s, a_d = _tc_mid(o3, b3.reshape(1, 64), g3.reshape(1, 64),
                           be3.reshape(1, 64), W4,
                           _amat(as4, 1, 64), _amat(ad4, 1, 64), 1)
    o4 = _GAT1(hw, a_s, a_d, srcs_p, dsts_p, offs2, zero64, zero1, neg1)
    pooled = _tc_pool(o4, b4.reshape(1, 64), g4.reshape(1, 64),
                      be4.reshape(1, 64), bat_p)
    v = _tc_final(pooled, Wf, bf.reshape(1, 128))
    return v[:G_SEG]
